# Initial kernel scaffold; baseline (speedup 1.0000x reference)
import jax
import jax.numpy as jnp
from jax.experimental import pallas as pl


def kernel(x, table):
    raise NotImplementedError("write your pallas kernel here")


# speedup vs baseline: 2.0241x; 2.0241x over previous
import jax
import jax.numpy as jnp
from jax.experimental import pallas as pl
from jax.experimental.pallas import tpu as pltpu
from jax.experimental.pallas import tpu_sc as plsc

_W = 128


def kernel(x, table):
    n = x.shape[0]
    d = table.shape[1]
    n_pad = -(-n // _W) * _W
    idx = x.reshape(n).astype(jnp.int32)
    idx = jnp.pad(idx, (0, n_pad - n)).reshape(1, n_pad)
    mesh = plsc.VectorSubcoreMesh(core_axis_name="c", subcore_axis_name="s")

    @pl.kernel(
        out_type=jax.ShapeDtypeStruct((n_pad, d), table.dtype),
        mesh=mesh,
    )
    def gather_kernel(table_hbm, idx_hbm, out_hbm):
        def body(idx_vmem, out_vmem):
            pltpu.sync_copy(table_hbm.at[idx_vmem.at[0]], out_vmem)

        pltpu.emit_pipeline(
            body,
            grid=(n_pad // _W,),
            in_specs=[pl.BlockSpec((1, _W), lambda i: (0, i))],
            out_specs=[pl.BlockSpec((_W, d), lambda i: (i, 0))],
            core_axis_name=("c", "s"),
            dimension_semantics=(pltpu.PARALLEL,),
        )(idx_hbm, out_hbm)

    return gather_kernel(table, idx)[:n]


# speedup vs baseline: 2.0246x; 1.0003x over previous
import jax
import jax.numpy as jnp
from jax.experimental import pallas as pl
from jax.experimental.pallas import tpu as pltpu
from jax.experimental.pallas import tpu_sc as plsc

_W = 128
_K = 2


def kernel(x, table):
    n = x.shape[0]
    d = table.shape[1]
    blk = _W * _K
    n_pad = -(-n // blk) * blk
    idx = jnp.pad(x.reshape(n).astype(jnp.int32), (0, n_pad - n))
    idx3 = idx.reshape(n_pad // blk, _K, _W)
    mesh = plsc.VectorSubcoreMesh(core_axis_name="c", subcore_axis_name="s")

    @pl.kernel(
        out_type=jax.ShapeDtypeStruct((n_pad, d), table.dtype),
        mesh=mesh,
    )
    def gather_kernel(table_hbm, idx_hbm, out_hbm):
        def body(idx_vmem, out_vmem):
            for j in range(_K):
                pltpu.sync_copy(
                    table_hbm.at[idx_vmem.at[0, j]],
                    out_vmem.at[pl.ds(j * _W, _W)],
                )

        pltpu.emit_pipeline(
            body,
            grid=(n_pad // blk,),
            in_specs=[pl.BlockSpec((1, _K, _W), lambda i: (i, 0, 0))],
            out_specs=[pl.BlockSpec((blk, d), lambda i: (i, 0))],
            core_axis_name=("c", "s"),
            dimension_semantics=(pltpu.PARALLEL,),
        )(idx_hbm, out_hbm)

    return gather_kernel(table, idx3)[:n]


# speedup vs baseline: 2.8960x; 1.4304x over previous
import jax
import jax.numpy as jnp
from jax import lax
from jax.experimental import pallas as pl
from jax.experimental.pallas import tpu as pltpu
from jax.experimental.pallas import tpu_sc as plsc

_W = 128
_NC = 2
_NS = 16
_NW = _NC * _NS


def kernel(x, table):
    n = x.shape[0]
    d = table.shape[1]
    idx = x.reshape(n).astype(jnp.int32)

    n_full = n // _W
    rem = n - n_full * _W
    n_win = n_full + (1 if rem else 0)
    wins_per_w = -(-n_win // _NW)
    per_w = wins_per_w * _W
    last_cnt = n - per_w * (_NW - 1)
    pairs = (wins_per_w + 1) // 2
    if rem:
        assert n_full // wins_per_w == _NW - 1 and rem % 8 == 0
    rem_rows = rem if rem else 8

    mesh = plsc.VectorSubcoreMesh(core_axis_name="c", subcore_axis_name="s")

    @pl.kernel(
        out_type=jax.ShapeDtypeStruct((n, d), table.dtype),
        mesh=mesh,
        scratch_types=[
            pltpu.VMEM((per_w,), jnp.int32),
            pltpu.VMEM((_W, d), table.dtype),
            pltpu.VMEM((_W, d), table.dtype),
            pltpu.VMEM((rem_rows, d), table.dtype),
            pltpu.SemaphoreType.DMA,
            pltpu.SemaphoreType.DMA,
        ],
    )
    def gather_kernel(table_hbm, idx_hbm, out_hbm, idx_v, buf_a, buf_b,
                      rem_buf, sem_a, sem_b):
        w = lax.axis_index("s") * _NC + lax.axis_index("c")
        full_mine = jnp.clip(n_full - wins_per_w * w, 0, wins_per_w)

        @pl.when(w < _NW - 1)
        def _():
            pltpu.sync_copy(idx_hbm.at[pl.ds(w * per_w, per_w)], idx_v)

        @pl.when(w == _NW - 1)
        def _():
            pltpu.sync_copy(
                idx_hbm.at[pl.ds((_NW - 1) * per_w, last_cnt)],
                idx_v.at[pl.ds(0, last_cnt)],
            )

        def issue_gather(j, buf, sem):
            pltpu.async_copy(
                table_hbm.at[idx_v.at[pl.ds(j * _W, _W)]], buf, sem)

        def wait_gather(buf, sem):
            pltpu.make_async_copy(out_hbm.at[pl.ds(0, _W)], buf, sem).wait()

        def write_out(j, buf):
            g = w * wins_per_w + j
            pltpu.sync_copy(buf, out_hbm.at[pl.ds(g * _W, _W)])

        @pl.when(full_mine > 0)
        def _():
            issue_gather(0, buf_a, sem_a)

        @pl.loop(0, pairs)
        def _(t):
            j0 = 2 * t
            j1 = j0 + 1

            @pl.when(j0 < full_mine)
            def _():
                @pl.when(j1 < full_mine)
                def _():
                    issue_gather(j1, buf_b, sem_b)

                wait_gather(buf_a, sem_a)
                write_out(j0, buf_a)

            @pl.when(j1 < full_mine)
            def _():
                @pl.when(j1 + 1 < full_mine)
                def _():
                    issue_gather(j1 + 1, buf_a, sem_a)

                wait_gather(buf_b, sem_b)
                write_out(j1, buf_b)

        if rem:
            @pl.when(w == _NW - 1)
            def _():
                lo = (n_full - wins_per_w * (_NW - 1)) * _W
                pltpu.async_copy(
                    table_hbm.at[idx_v.at[pl.ds(lo, rem)]], rem_buf,
                    sem_a).wait()
                pltpu.sync_copy(
                    rem_buf, out_hbm.at[pl.ds(n_full * _W, rem)])

    return gather_kernel(table, idx)
